# baseline (device time: 222887 ns/iter reference)
import jax
import jax.numpy as jnp
from jax import lax
from jax.experimental import pallas as pl
from jax.experimental.pallas import tpu as pltpu

N_DEV = 16
G = 2


def kernel(x):
    m, n = x.shape
    Qr = m // 4
    qg = Qr // G
    hr = qg // 2
    qs = qg // 4

    def body(x_ref, out_ref, aL0, aS, aF, bL0, bS, bF, pr, uin, din,
             pa_ssem, pa_rsem, u_ssem, u_rsem, dn_ssem, dn_rsem,
             uag_ssem, uag_rsem, dag_ssem, dag_rsem, pc_ssem, pc_rsem):
        d = lax.axis_index("i")
        p = lax.div(d, 4)
        j = lax.rem(d, 4)

        def jj(off):
            return lax.rem(j + off + 8, 4)

        jl = 4 * p + lax.rem(j + 3, 4)
        jr = 4 * p + lax.rem(j + 1, 4)
        cup = lax.rem(d + 4, N_DEV)
        cdn = lax.rem(d + 12, N_DEV)

        MESH = pl.DeviceIdType.MESH

        def copy(src, dst, ssem, rsem, dev):
            return pltpu.make_async_remote_copy(
                src_ref=src, dst_ref=dst, send_sem=ssem, recv_sem=rsem,
                device_id=(dev,), device_id_type=MESH,
            )

        def xhalf(off, half, g):
            return x_ref.at[pl.ds(jj(off) * Qr + g * qg + half * hr, hr), :]

        def prsub(g, q):
            return pr.at[g, pl.ds(q * qs, qs), :]

        def outsub(g, q):
            return out_ref.at[pl.ds(j * Qr + g * qg + q * qs, qs), :]

        def outhalf(off, half, g):
            return out_ref.at[pl.ds(jj(off) * Qr + g * qg + half * hr, hr), :]

        def A0(g):
            return copy(xhalf(-2, 0, g), aL0.at[g], pa_ssem.at[g, 0],
                        pa_rsem.at[g, 0], jl)

        def A0s(g):
            return copy(xhalf(1, 0, g), aS.at[g], pa_ssem.at[g, 1],
                        pa_rsem.at[g, 1], jr)

        def A1(g):
            return copy(aL0.at[g], aF.at[g], pa_ssem.at[g, 2],
                        pa_rsem.at[g, 2], jl)

        def B0(g):
            return copy(xhalf(2, 1, g), bL0.at[g], pa_ssem.at[g, 3],
                        pa_rsem.at[g, 3], jr)

        def B0s(g):
            return copy(xhalf(-1, 1, g), bS.at[g], pa_ssem.at[g, 4],
                        pa_rsem.at[g, 4], jl)

        def B1(g):
            return copy(bL0.at[g], bF.at[g], pa_ssem.at[g, 5],
                        pa_rsem.at[g, 5], jr)

        def u_raw(g, q):
            return copy(prsub(g, q), uin.at[g, q], u_ssem.at[g, q],
                        u_rsem.at[g, q], cup)

        def u_acc(g, q):
            return copy(uin.at[g, q], uin.at[g, q], u_ssem.at[g, q],
                        u_rsem.at[g, q], cup)

        def d_raw(g, q):
            return copy(prsub(g, q), din.at[g, q], dn_ssem.at[g, q],
                        dn_rsem.at[g, q], cdn)

        def d_acc(g, q):
            return copy(din.at[g, q], din.at[g, q], dn_ssem.at[g, q],
                        dn_rsem.at[g, q], cdn)

        def uag(g, q):
            return copy(outsub(g, q), outsub(g, q), uag_ssem.at[g, q],
                        uag_rsem.at[g, q], cup)

        def dag(g, q):
            return copy(outsub(g, q), outsub(g, q), dag_ssem.at[g, q],
                        dag_rsem.at[g, q], cdn)

        def cA1(g):
            return copy(outhalf(0, 0, g), outhalf(0, 0, g),
                        pc_ssem.at[g, 0], pc_rsem.at[g, 0], jr)

        def cAs(g):
            return copy(outhalf(0, 0, g), outhalf(0, 0, g),
                        pc_ssem.at[g, 1], pc_rsem.at[g, 1], jl)

        def cB1(g):
            return copy(outhalf(0, 1, g), outhalf(0, 1, g),
                        pc_ssem.at[g, 2], pc_rsem.at[g, 2], jl)

        def cBs(g):
            return copy(outhalf(0, 1, g), outhalf(0, 1, g),
                        pc_ssem.at[g, 3], pc_rsem.at[g, 3], jr)

        def cA2(g):
            return copy(outhalf(-1, 0, g), outhalf(-1, 0, g),
                        pc_ssem.at[g, 4], pc_rsem.at[g, 4], jr)

        def cB2(g):
            return copy(outhalf(1, 1, g), outhalf(1, 1, g),
                        pc_ssem.at[g, 5], pc_rsem.at[g, 5], jl)

        barrier_sem = pltpu.get_barrier_semaphore()
        for nbr in (jl, jr):
            pl.semaphore_signal(barrier_sem, inc=1, device_id=(nbr,),
                                device_id_type=MESH)

        @pl.when(p < 3)
        def _():
            pl.semaphore_signal(barrier_sem, inc=1, device_id=(cup,),
                                device_id_type=MESH)

        @pl.when(p > 0)
        def _():
            pl.semaphore_signal(barrier_sem, inc=1, device_id=(cdn,),
                                device_id_type=MESH)

        pl.semaphore_wait(barrier_sem, 2)

        @pl.when(p > 0)
        def _():
            pl.semaphore_wait(barrier_sem, 1)

        @pl.when(p < 3)
        def _():
            pl.semaphore_wait(barrier_sem, 1)

        for g in range(G):
            A0(g).start()
            A0s(g).start()
            B0(g).start()
            B0s(g).start()

        for g in range(G):
            A0(g).wait_recv()
            aL0[g, :, :] = aL0[g, :, :] + xhalf(-1, 0, g)[:, :]
            A1(g).start()
            B0(g).wait_recv()
            bL0[g, :, :] = bL0[g, :, :] + xhalf(1, 1, g)[:, :]
            B1(g).start()

            A0s(g).wait_recv()
            A1(g).wait_recv()
            pr[g, pl.ds(0, hr), :] = (
                aF[g, :, :] + aS[g, :, :] + xhalf(0, 0, g)[:, :]
            )
            B0s(g).wait_recv()
            B1(g).wait_recv()
            pr[g, pl.ds(hr, hr), :] = (
                bF[g, :, :] + bS[g, :, :] + xhalf(0, 1, g)[:, :]
            )

            def ublock(q, g=g):
                @pl.when(p == 0)
                def _():
                    u_raw(g, q).start()

                @pl.when((p >= 1) & (p <= q))
                def _():
                    u_acc(g, q).wait_recv()
                    uin[g, q, :, :] = uin[g, q, :, :] + prsub(g, q)[:, :]

                @pl.when((p >= 1) & (p < q))
                def _():
                    u_acc(g, q).start()

            def dblock(q, g=g):
                @pl.when(p == 3)
                def _():
                    d_raw(g, q).start()

                @pl.when((p >= q) & (p <= 2))
                def _():
                    d_acc(g, q).wait_recv()

                @pl.when((p >= q + 1) & (p <= 2))
                def _():
                    din[g, q, :, :] = din[g, q, :, :] + prsub(g, q)[:, :]
                    d_acc(g, q).start()

            ublock(3)
            dblock(0)
            ublock(2)
            dblock(1)
            ublock(1)
            dblock(2)

            for q in range(4):
                @pl.when(p == q)
                def _(q=q, g=g):
                    if q == 0:
                        val = prsub(g, 0)[:, :] + din[g, 0, :, :]
                    elif q == 3:
                        val = uin[g, 3, :, :]
                    else:
                        val = uin[g, q, :, :] + din[g, q, :, :]
                    out_ref[pl.ds(j * Qr + g * qg + q * qs, qs), :] = val

                @pl.when(p > q)
                def _(q=q, g=g):
                    uag(g, q).wait_recv()

                @pl.when((p >= q) & (p <= 2))
                def _(q=q, g=g):
                    uag(g, q).start()

                @pl.when(p < q)
                def _(q=q, g=g):
                    dag(g, q).wait_recv()

                @pl.when((p <= q) & (p >= 1))
                def _(q=q, g=g):
                    dag(g, q).start()

            cA1(g).start()
            cAs(g).start()
            cB1(g).start()
            cBs(g).start()

        for g in range(G):
            cA1(g).wait_recv()
            cA2(g).start()
            cB1(g).wait_recv()
            cB2(g).start()
            cAs(g).wait_recv()
            cBs(g).wait_recv()
            cA2(g).wait_recv()
            cB2(g).wait_recv()

        for g in range(G):
            for desc in (A0(g), A0s(g), A1(g), B0(g), B0s(g), B1(g),
                         cA1(g), cAs(g), cB1(g), cBs(g), cA2(g), cB2(g)):
                desc.wait_send()
            for q in range(1, 4):
                @pl.when(p == 0)
                def _(q=q, g=g):
                    u_raw(g, q).wait_send()

                @pl.when((p >= 1) & (p < q))
                def _(q=q, g=g):
                    u_acc(g, q).wait_send()

            for q in range(3):
                @pl.when(p == 3)
                def _(q=q, g=g):
                    d_raw(g, q).wait_send()

                @pl.when((p >= q + 1) & (p <= 2))
                def _(q=q, g=g):
                    d_acc(g, q).wait_send()

            for q in range(4):
                @pl.when((p >= q) & (p <= 2))
                def _(q=q, g=g):
                    uag(g, q).wait_send()

                @pl.when((p <= q) & (p >= 1))
                def _(q=q, g=g):
                    dag(g, q).wait_send()

    return pl.pallas_call(
        body,
        out_shape=jax.ShapeDtypeStruct((m, n), x.dtype),
        in_specs=[pl.BlockSpec(memory_space=pltpu.VMEM)],
        out_specs=pl.BlockSpec(memory_space=pltpu.VMEM),
        scratch_shapes=[
            pltpu.VMEM((G, hr, n), x.dtype),
            pltpu.VMEM((G, hr, n), x.dtype),
            pltpu.VMEM((G, hr, n), x.dtype),
            pltpu.VMEM((G, hr, n), x.dtype),
            pltpu.VMEM((G, hr, n), x.dtype),
            pltpu.VMEM((G, hr, n), x.dtype),
            pltpu.VMEM((G, qg, n), x.dtype),
            pltpu.VMEM((G, 4, qs, n), x.dtype),
            pltpu.VMEM((G, 4, qs, n), x.dtype),
            pltpu.SemaphoreType.DMA((G, 6)),
            pltpu.SemaphoreType.DMA((G, 6)),
            pltpu.SemaphoreType.DMA((G, 4)),
            pltpu.SemaphoreType.DMA((G, 4)),
            pltpu.SemaphoreType.DMA((G, 4)),
            pltpu.SemaphoreType.DMA((G, 4)),
            pltpu.SemaphoreType.DMA((G, 4)),
            pltpu.SemaphoreType.DMA((G, 4)),
            pltpu.SemaphoreType.DMA((G, 4)),
            pltpu.SemaphoreType.DMA((G, 4)),
            pltpu.SemaphoreType.DMA((G, 6)),
            pltpu.SemaphoreType.DMA((G, 6)),
        ],
        compiler_params=pltpu.CompilerParams(collective_id=0),
    )(x)


# device time: 130984 ns/iter; 1.7016x vs baseline; 1.7016x over previous
import jax
import jax.numpy as jnp
from jax import lax
from jax.experimental import pallas as pl
from jax.experimental.pallas import tpu as pltpu

N_DEV = 16
G = 2


def kernel(x):
    m, n = x.shape
    Qr = m // 4
    qg = Qr // G
    hr = qg // 2
    qs = qg // 4
    hq = qs // 2

    def body(x_ref, out_ref,
             aL0, aS, aF, bL0, bS, bF, pr,
             rL0, rS, rF, sL0, sS, sF,
             pa_s, pa_r, pb_s, pb_r, pg_s, pg_r, pc_s, pc_r):
        d = lax.axis_index("i")
        p = lax.div(d, 4)
        j = lax.rem(d, 4)

        def jj(off):
            return lax.rem(j + off + 8, 4)

        def pp(off):
            return lax.rem(p + off + 8, 4)

        jl = 4 * p + lax.rem(j + 3, 4)
        jr = 4 * p + lax.rem(j + 1, 4)
        cdn = lax.rem(d + 12, N_DEV)
        cup = lax.rem(d + 4, N_DEV)

        MESH = pl.DeviceIdType.MESH

        def copy(src, dst, ssem, rsem, dev):
            return pltpu.make_async_remote_copy(
                src_ref=src, dst_ref=dst, send_sem=ssem, recv_sem=rsem,
                device_id=(dev,), device_id_type=MESH,
            )

        def xhalf(off, half, g):
            return x_ref.at[pl.ds(jj(off) * Qr + g * qg + half * hr, hr), :]

        def prhalf(off, half, g):
            return pr.at[g, pl.ds(pp(off) * qs + half * hq, hq), :]

        def breg(off, half, g):
            return out_ref.at[
                pl.ds(j * Qr + g * qg + pp(off) * qs + half * hq, hq), :
            ]

        def creg(off, half, g):
            return out_ref.at[
                pl.ds(jj(off) * Qr + g * qg + half * hr, hr), :
            ]

        def rs_descs(g, slicer, bufs, ssem, rsem, dl, dr):
            h0, h0s, h0f, h1, h1s, h1f = bufs
            return {
                "L0": copy(slicer(-2, 0, g), h0.at[g], ssem.at[g, 0],
                           rsem.at[g, 0], dl),
                "L0s": copy(slicer(1, 0, g), h0s.at[g], ssem.at[g, 1],
                            rsem.at[g, 1], dr),
                "L1": copy(h0.at[g], h0f.at[g], ssem.at[g, 2],
                           rsem.at[g, 2], dl),
                "M0": copy(slicer(2, 1, g), h1.at[g], ssem.at[g, 3],
                           rsem.at[g, 3], dr),
                "M0s": copy(slicer(-1, 1, g), h1s.at[g], ssem.at[g, 4],
                            rsem.at[g, 4], dl),
                "M1": copy(h1.at[g], h1f.at[g], ssem.at[g, 5],
                           rsem.at[g, 5], dr),
            }

        def ag_descs(g, region, ssem, rsem, dl, dr):
            return {
                "A1": copy(region(0, 0, g), region(0, 0, g),
                           ssem.at[g, 0], rsem.at[g, 0], dr),
                "As": copy(region(0, 0, g), region(0, 0, g),
                           ssem.at[g, 1], rsem.at[g, 1], dl),
                "B1": copy(region(0, 1, g), region(0, 1, g),
                           ssem.at[g, 2], rsem.at[g, 2], dl),
                "Bs": copy(region(0, 1, g), region(0, 1, g),
                           ssem.at[g, 3], rsem.at[g, 3], dr),
                "A2": copy(region(-1, 0, g), region(-1, 0, g),
                           ssem.at[g, 4], rsem.at[g, 4], dr),
                "B2": copy(region(1, 1, g), region(1, 1, g),
                           ssem.at[g, 5], rsem.at[g, 5], dl),
            }

        abufs = (aL0, aS, aF, bL0, bS, bF)
        bbufs = (rL0, rS, rF, sL0, sS, sF)

        def A(g):
            return rs_descs(g, xhalf, abufs, pa_s, pa_r, jl, jr)

        def B(g):
            return rs_descs(g, prhalf, bbufs, pb_s, pb_r, cdn, cup)

        def BAG(g):
            return ag_descs(g, breg, pg_s, pg_r, cdn, cup)

        def C(g):
            return ag_descs(g, creg, pc_s, pc_r, jl, jr)

        barrier_sem = pltpu.get_barrier_semaphore()
        for nbr in (jl, jr, cdn, cup):
            pl.semaphore_signal(barrier_sem, inc=1, device_id=(nbr,),
                                device_id_type=MESH)
        pl.semaphore_wait(barrier_sem, 4)

        for g in range(G):
            a = A(g)
            a["L0"].start()
            a["L0s"].start()
            a["M0"].start()
            a["M0s"].start()

        for g in range(G):
            a = A(g)
            a["L0"].wait_recv()
            aL0[g, :, :] = aL0[g, :, :] + xhalf(-1, 0, g)[:, :]
            a["L1"].start()
            a["M0"].wait_recv()
            bL0[g, :, :] = bL0[g, :, :] + xhalf(1, 1, g)[:, :]
            a["M1"].start()
            a["L0s"].wait_recv()
            a["L1"].wait_recv()
            pr[g, pl.ds(0, hr), :] = (
                aF[g, :, :] + aS[g, :, :] + xhalf(0, 0, g)[:, :]
            )
            a["M0s"].wait_recv()
            a["M1"].wait_recv()
            pr[g, pl.ds(hr, hr), :] = (
                bF[g, :, :] + bS[g, :, :] + xhalf(0, 1, g)[:, :]
            )

            b = B(g)
            b["L0"].start()
            b["L0s"].start()
            b["M0"].start()
            b["M0s"].start()
            b["L0"].wait_recv()
            rL0[g, :, :] = rL0[g, :, :] + prhalf(-1, 0, g)[:, :]
            b["L1"].start()
            b["M0"].wait_recv()
            sL0[g, :, :] = sL0[g, :, :] + prhalf(1, 1, g)[:, :]
            b["M1"].start()
            b["L0s"].wait_recv()
            b["L1"].wait_recv()
            out_ref[pl.ds(j * Qr + g * qg + p * qs, hq), :] = (
                rF[g, :, :] + rS[g, :, :] + prhalf(0, 0, g)[:, :]
            )
            b["M0s"].wait_recv()
            b["M1"].wait_recv()
            out_ref[pl.ds(j * Qr + g * qg + p * qs + hq, hq), :] = (
                sF[g, :, :] + sS[g, :, :] + prhalf(0, 1, g)[:, :]
            )

            bg = BAG(g)
            bg["A1"].start()
            bg["As"].start()
            bg["B1"].start()
            bg["Bs"].start()
            bg["A1"].wait_recv()
            bg["A2"].start()
            bg["B1"].wait_recv()
            bg["B2"].start()
            bg["As"].wait_recv()
            bg["Bs"].wait_recv()
            bg["A2"].wait_recv()
            bg["B2"].wait_recv()

            c = C(g)
            c["A1"].start()
            c["As"].start()
            c["B1"].start()
            c["Bs"].start()

        for g in range(G):
            c = C(g)
            c["A1"].wait_recv()
            c["A2"].start()
            c["B1"].wait_recv()
            c["B2"].start()
            c["As"].wait_recv()
            c["Bs"].wait_recv()
            c["A2"].wait_recv()
            c["B2"].wait_recv()

        for g in range(G):
            for dset in (A(g), B(g)):
                for k in ("L0", "L0s", "L1", "M0", "M0s", "M1"):
                    dset[k].wait_send()
            for dset in (BAG(g), C(g)):
                for k in ("A1", "As", "B1", "Bs", "A2", "B2"):
                    dset[k].wait_send()

    return pl.pallas_call(
        body,
        out_shape=jax.ShapeDtypeStruct((m, n), x.dtype),
        in_specs=[pl.BlockSpec(memory_space=pltpu.VMEM)],
        out_specs=pl.BlockSpec(memory_space=pltpu.VMEM),
        scratch_shapes=[
            pltpu.VMEM((G, hr, n), x.dtype),
            pltpu.VMEM((G, hr, n), x.dtype),
            pltpu.VMEM((G, hr, n), x.dtype),
            pltpu.VMEM((G, hr, n), x.dtype),
            pltpu.VMEM((G, hr, n), x.dtype),
            pltpu.VMEM((G, hr, n), x.dtype),
            pltpu.VMEM((G, qg, n), x.dtype),
            pltpu.VMEM((G, hq, n), x.dtype),
            pltpu.VMEM((G, hq, n), x.dtype),
            pltpu.VMEM((G, hq, n), x.dtype),
            pltpu.VMEM((G, hq, n), x.dtype),
            pltpu.VMEM((G, hq, n), x.dtype),
            pltpu.VMEM((G, hq, n), x.dtype),
            pltpu.SemaphoreType.DMA((G, 6)),
            pltpu.SemaphoreType.DMA((G, 6)),
            pltpu.SemaphoreType.DMA((G, 6)),
            pltpu.SemaphoreType.DMA((G, 6)),
            pltpu.SemaphoreType.DMA((G, 6)),
            pltpu.SemaphoreType.DMA((G, 6)),
            pltpu.SemaphoreType.DMA((G, 6)),
            pltpu.SemaphoreType.DMA((G, 6)),
        ],
        compiler_params=pltpu.CompilerParams(collective_id=0),
    )(x)
